# Initial kernel scaffold; baseline (speedup 1.0000x reference)
import jax
import jax.numpy as jnp
from jax.experimental import pallas as pl


def kernel(x, log_w, tau_s):
    raise NotImplementedError("write your pallas kernel here")


# speedup vs baseline: 216.1390x; 216.1390x over previous
import functools

import jax
import jax.numpy as jnp
from jax import lax
from jax.experimental import pallas as pl
from jax.experimental.pallas import tpu as pltpu
from jax.experimental.pallas import tpu_sc as plsc

N = 16384
C = 512
ROW = 2 * C
NW = 32
ROWS_PER_W = N // NW
CHUNK = 32
NCHUNK = ROWS_PER_W // CHUNK
TBL = 64 * 64


def _body(x_hbm, lw_hbm, rtau_hbm, out_hbm,
          lw_v, rtau_v, tab_v, xbuf, out_v, sem0, sem1):
    nc = 2
    wid = lax.axis_index("s") * nc + lax.axis_index("c")
    row0 = wid * ROWS_PER_W

    pltpu.sync_copy(lw_hbm, lw_v)
    pltpu.sync_copy(rtau_hbm, rtau_v)
    rtau = rtau_v[...]

    @pl.loop(0, TBL // 16)
    def _build(i):
        base = i * 16
        idx = base + lax.iota(jnp.int32, 16)
        a = idx >> 6
        b = idx & 63
        d = (a - b) & 127
        lw = plsc.load_gather(lw_v, [d])
        t = jnp.maximum(a, b).astype(jnp.float32)
        tab_v[pl.ds(base, 16)] = jnp.exp(lw - (2.0 - t) * rtau)

    lane = lax.iota(jnp.int32, 16)

    def compute(buf, chunk):
        for g in range(CHUNK // 16):
            ridx = g * 16 + lane

            @pl.loop(0, C, init_carry=jnp.zeros((16,), jnp.float32), unroll=4)
            def _inner(c, acc):
                cb = lax.broadcast(c, (16,))
                v0 = plsc.load_gather(buf, [ridx, cb])
                v1 = plsc.load_gather(buf, [ridx, cb + C])
                a = ((1.0 - v0) * 63.0).astype(jnp.int32)
                b = ((1.0 - v1) * 63.0).astype(jnp.int32)
                return acc + plsc.load_gather(tab_v, [(a << 6) | b])

            out_v[pl.ds(chunk * CHUNK + g * 16, 16)] = _inner

    def start(i, buf_i, sem):
        return pltpu.async_copy(
            x_hbm.at[pl.ds(row0 + i * CHUNK, CHUNK)], xbuf.at[buf_i], sem)

    start(0, 0, sem0).wait()
    for i in range(NCHUNK):
        nxt = None
        if i + 1 < NCHUNK:
            nxt = start(i + 1, (i + 1) % 2, sem1 if (i + 1) % 2 else sem0)
        compute(xbuf.at[i % 2], i)
        if nxt is not None:
            nxt.wait()

    pltpu.sync_copy(out_v, out_hbm.at[pl.ds(row0, ROWS_PER_W)])


@jax.jit
def kernel(x, log_w, tau_s):
    mesh = plsc.VectorSubcoreMesh(core_axis_name="c", subcore_axis_name="s")
    run = functools.partial(
        pl.kernel,
        mesh=mesh,
        compiler_params=pltpu.CompilerParams(needs_layout_passes=False),
        out_type=jax.ShapeDtypeStruct((N,), jnp.float32),
        scratch_types=[
            pltpu.VMEM((128,), jnp.float32),
            pltpu.VMEM((16,), jnp.float32),
            pltpu.VMEM((TBL,), jnp.float32),
            pltpu.VMEM((2, CHUNK, ROW), jnp.float32),
            pltpu.VMEM((ROWS_PER_W,), jnp.float32),
            pltpu.SemaphoreType.DMA,
            pltpu.SemaphoreType.DMA,
        ],
    )(_body)
    x2 = x.reshape(N, ROW)
    rtau = jnp.full((16,), 1.0, jnp.float32) / tau_s
    out = run(x2, log_w, rtau)
    return out.reshape(N, 1)


# speedup vs baseline: 518.8506x; 2.4005x over previous
import functools

import jax
import jax.numpy as jnp
from jax import lax
from jax.experimental import pallas as pl
from jax.experimental.pallas import tpu as pltpu
from jax.experimental.pallas import tpu_sc as plsc

N = 16384
C = 512
ROW = 2 * C
NW = 32
ROWS_PER_W = N // NW
CHUNK = 32
NCHUNK = ROWS_PER_W // CHUNK
TBL = 64 * 64


def _body(x_hbm, lw_hbm, rtau_hbm, out_hbm,
          lw_v, rtau_v, tab_v, xbuf, out_v, sem0, sem1):
    nc = 2
    wid = lax.axis_index("s") * nc + lax.axis_index("c")
    row0 = wid * ROWS_PER_W

    pltpu.sync_copy(lw_hbm, lw_v)
    pltpu.sync_copy(rtau_hbm, rtau_v)
    rtau = rtau_v[...]

    @pl.loop(0, TBL // 16)
    def _build(i):
        base = i * 16
        idx = base + lax.iota(jnp.int32, 16)
        a = idx >> 6
        b = idx & 63
        d = (a - b) & 127
        lw = plsc.load_gather(lw_v, [d])
        t = jnp.maximum(a, b).astype(jnp.float32)
        tab_v[pl.ds(base, 16)] = jnp.exp(lw - (2.0 - t) * rtau)

    lane = lax.iota(jnp.int32, 16)
    lane17 = lane * 17

    def compute(buf, chunk):
        for g in range(CHUNK // 16):
            ridx = g * 16 + lane

            @pl.loop(0, C, init_carry=jnp.zeros((16,), jnp.float32), unroll=4)
            def _inner(c, acc):
                j0 = (lane17 + c) & (C - 1)
                v0 = plsc.load_gather(buf, [ridx, j0])
                v1 = plsc.load_gather(buf, [ridx, j0 | C])
                a = ((1.0 - v0) * 63.0).astype(jnp.int32)
                b = ((1.0 - v1) * 63.0).astype(jnp.int32)
                return acc + plsc.load_gather(tab_v, [(a << 6) | b])

            out_v[pl.ds(chunk * CHUNK + g * 16, 16)] = _inner

    def start(i, buf_i, sem):
        return pltpu.async_copy(
            x_hbm.at[pl.ds(row0 + i * CHUNK, CHUNK)], xbuf.at[buf_i], sem)

    start(0, 0, sem0).wait()
    for i in range(NCHUNK):
        nxt = None
        if i + 1 < NCHUNK:
            nxt = start(i + 1, (i + 1) % 2, sem1 if (i + 1) % 2 else sem0)
        compute(xbuf.at[i % 2], i)
        if nxt is not None:
            nxt.wait()

    pltpu.sync_copy(out_v, out_hbm.at[pl.ds(row0, ROWS_PER_W)])


@jax.jit
def kernel(x, log_w, tau_s):
    mesh = plsc.VectorSubcoreMesh(core_axis_name="c", subcore_axis_name="s")
    run = functools.partial(
        pl.kernel,
        mesh=mesh,
        compiler_params=pltpu.CompilerParams(needs_layout_passes=False),
        out_type=jax.ShapeDtypeStruct((N,), jnp.float32),
        scratch_types=[
            pltpu.VMEM((128,), jnp.float32),
            pltpu.VMEM((16,), jnp.float32),
            pltpu.VMEM((TBL,), jnp.float32),
            pltpu.VMEM((2, CHUNK, ROW), jnp.float32),
            pltpu.VMEM((ROWS_PER_W,), jnp.float32),
            pltpu.SemaphoreType.DMA,
            pltpu.SemaphoreType.DMA,
        ],
    )(_body)
    x2 = x.reshape(N, ROW)
    rtau = jnp.full((16,), 1.0, jnp.float32) / tau_s
    out = run(x2, log_w, rtau)
    return out.reshape(N, 1)


# speedup vs baseline: 532.6585x; 1.0266x over previous
import functools

import jax
import jax.numpy as jnp
from jax import lax
from jax.experimental import pallas as pl
from jax.experimental.pallas import tpu as pltpu
from jax.experimental.pallas import tpu_sc as plsc

N = 16384
C = 512
ROW = 2 * C
NW = 32
ROWS_PER_W = N // NW
CHUNK = 16
NCHUNK = ROWS_PER_W // CHUNK
TBL = 64 * 64


def _body(x_hbm, lw_hbm, rtau_hbm, out_hbm,
          lw_v, rtau_v, tab_v, red_v, xbuf0, xbuf1, out_v, sem0, sem1):
    nc = 2
    wid = lax.axis_index("s") * nc + lax.axis_index("c")
    row0 = wid * ROWS_PER_W

    pltpu.sync_copy(lw_hbm, lw_v)
    pltpu.sync_copy(rtau_hbm, rtau_v)
    rtau = rtau_v[...]

    lane = lax.iota(jnp.int32, 16)
    lane16 = lane * 16
    lane17 = lane * 17
    splats = [jnp.full((16,), k, jnp.int32) for k in range(16)]

    @pl.loop(0, TBL // 16)
    def _build(i):
        base = i * 16
        idx = base + lane
        a = idx >> 6
        b = idx & 63
        d = (a - b) & 127
        lw = plsc.load_gather(lw_v, [d])
        t = jnp.maximum(a, b).astype(jnp.float32)
        val = jnp.exp(lw - (2.0 - t) * rtau)
        for k in range(16):
            tab_v[pl.ds((base + k) * 16, 16)] = jnp.take(val, splats[k])

    def phase(ci, buf, sem):
        src = x_hbm.at[pl.ds((row0 + ci * CHUNK) * ROW, CHUNK * ROW)]
        pltpu.make_async_copy(src, buf, sem).wait()

        @pl.loop(0, CHUNK)
        def _rows(r):
            rbase = r * ROW

            @pl.loop(0, C // 16,
                     init_carry=jnp.zeros((16,), jnp.float32), unroll=4)
            def _inner(cc, acc):
                v0 = buf[pl.ds(rbase + cc * 16, 16)]
                v1 = buf[pl.ds(rbase + C + cc * 16, 16)]
                a = ((1.0 - v0) * 63.0).astype(jnp.int32)
                b = ((1.0 - v1) * 63.0).astype(jnp.int32)
                j = ((a << 10) | (b << 4)) | lane
                return acc + plsc.load_gather(tab_v, [j])

            red_v[pl.ds(r * 17, 16)] = _inner

        tot = jnp.zeros((16,), jnp.float32)
        for l in range(16):
            tot = tot + plsc.load_gather(red_v, [lane17 + l])
        out_v[pl.ds(ci * CHUNK, 16)] = tot

        @pl.when(ci + 2 < NCHUNK)
        def _():
            nsrc = x_hbm.at[pl.ds((row0 + (ci + 2) * CHUNK) * ROW,
                                  CHUNK * ROW)]
            pltpu.async_copy(nsrc, buf, sem)

    pltpu.async_copy(
        x_hbm.at[pl.ds(row0 * ROW, CHUNK * ROW)], xbuf0, sem0)
    pltpu.async_copy(
        x_hbm.at[pl.ds((row0 + CHUNK) * ROW, CHUNK * ROW)], xbuf1, sem1)

    @pl.loop(0, NCHUNK, step=2)
    def _chunks(i):
        phase(i, xbuf0, sem0)
        phase(i + 1, xbuf1, sem1)

    pltpu.sync_copy(out_v, out_hbm.at[pl.ds(row0, ROWS_PER_W)])


@jax.jit
def kernel(x, log_w, tau_s):
    mesh = plsc.VectorSubcoreMesh(core_axis_name="c", subcore_axis_name="s")
    run = functools.partial(
        pl.kernel,
        mesh=mesh,
        compiler_params=pltpu.CompilerParams(needs_layout_passes=False),
        out_type=jax.ShapeDtypeStruct((N,), jnp.float32),
        scratch_types=[
            pltpu.VMEM((128,), jnp.float32),
            pltpu.VMEM((16,), jnp.float32),
            pltpu.VMEM((TBL * 16,), jnp.float32),
            pltpu.VMEM((CHUNK * 17,), jnp.float32),
            pltpu.VMEM((CHUNK * ROW,), jnp.float32),
            pltpu.VMEM((CHUNK * ROW,), jnp.float32),
            pltpu.VMEM((ROWS_PER_W,), jnp.float32),
            pltpu.SemaphoreType.DMA,
            pltpu.SemaphoreType.DMA,
        ],
    )(_body)
    xf = x.reshape(N * ROW)
    rtau = jnp.full((16,), 1.0, jnp.float32) / tau_s
    out = run(xf, log_w, rtau)
    return out.reshape(N, 1)


# speedup vs baseline: 882.5440x; 1.6569x over previous
import functools

import jax
import jax.numpy as jnp
from jax import lax
from jax.experimental import pallas as pl
from jax.experimental.pallas import tpu as pltpu
from jax.experimental.pallas import tpu_sc as plsc

N = 16384
C = 512
ROW = 2 * C
NW = 32
ROWS_PER_W = N // NW
CHUNK = 16
NCHUNK = ROWS_PER_W // CHUNK
TBL = 64 * 64


def _body(x_hbm, lw_hbm, rtau_hbm, out_hbm,
          lw_v, rtau_v, tab_v, red_v, xbuf0, xbuf1, out_v, sem0, sem1):
    nc = 2
    wid = lax.axis_index("s") * nc + lax.axis_index("c")
    row0 = wid * ROWS_PER_W

    pltpu.sync_copy(lw_hbm, lw_v)
    pltpu.sync_copy(rtau_hbm, rtau_v)
    rtau = rtau_v[...]

    lane = lax.iota(jnp.int32, 16)
    lane16 = lane * 16
    lane17 = lane * 17
    splats = [jnp.full((16,), k, jnp.int32) for k in range(16)]

    @pl.loop(0, TBL // 16)
    def _build(i):
        base = i * 16
        idx = base + lane
        a = idx >> 6
        b = idx & 63
        d = (a - b) & 127
        lw = plsc.load_gather(lw_v, [d])
        t = jnp.maximum(a, b).astype(jnp.float32)
        val = jnp.exp(lw - (2.0 - t) * rtau)
        for k in range(16):
            tab_v[pl.ds((base + k) * 16, 16)] = jnp.take(val, splats[k])

    def phase(ci, buf, sem):
        src = x_hbm.at[pl.ds(row0 + ci * CHUNK, CHUNK)]
        pltpu.make_async_copy(src, buf, sem).wait()

        @pl.loop(0, CHUNK)
        def _rows(r):

            @pl.loop(0, C // 16,
                     init_carry=jnp.zeros((16,), jnp.float32), unroll=4)
            def _inner(cc, acc):
                v0 = buf[r, 0, pl.ds(cc * 16, 16)]
                v1 = buf[r, 1, pl.ds(cc * 16, 16)]
                a = ((1.0 - v0) * 63.0).astype(jnp.int32)
                b = ((1.0 - v1) * 63.0).astype(jnp.int32)
                j = ((a << 10) | (b << 4)) | lane
                return acc + plsc.load_gather(tab_v, [j])

            red_v[pl.ds(r * 17, 16)] = _inner

        tot = jnp.zeros((16,), jnp.float32)
        for l in range(16):
            tot = tot + plsc.load_gather(red_v, [lane17 + l])
        out_v[pl.ds(ci * CHUNK, 16)] = tot

        @pl.when(ci + 2 < NCHUNK)
        def _():
            nsrc = x_hbm.at[pl.ds(row0 + (ci + 2) * CHUNK, CHUNK)]
            pltpu.async_copy(nsrc, buf, sem)

    pltpu.async_copy(x_hbm.at[pl.ds(row0, CHUNK)], xbuf0, sem0)
    pltpu.async_copy(x_hbm.at[pl.ds(row0 + CHUNK, CHUNK)], xbuf1, sem1)

    @pl.loop(0, NCHUNK, step=2)
    def _chunks(i):
        phase(i, xbuf0, sem0)
        phase(i + 1, xbuf1, sem1)

    pltpu.sync_copy(out_v, out_hbm.at[pl.ds(row0, ROWS_PER_W)])


@jax.jit
def kernel(x, log_w, tau_s):
    mesh = plsc.VectorSubcoreMesh(core_axis_name="c", subcore_axis_name="s")
    run = functools.partial(
        pl.kernel,
        mesh=mesh,
        compiler_params=pltpu.CompilerParams(needs_layout_passes=False),
        out_type=jax.ShapeDtypeStruct((N,), jnp.float32),
        scratch_types=[
            pltpu.VMEM((128,), jnp.float32),
            pltpu.VMEM((16,), jnp.float32),
            pltpu.VMEM((TBL * 16,), jnp.float32),
            pltpu.VMEM((CHUNK * 17,), jnp.float32),
            pltpu.VMEM((CHUNK, 2, C), jnp.float32),
            pltpu.VMEM((CHUNK, 2, C), jnp.float32),
            pltpu.VMEM((ROWS_PER_W,), jnp.float32),
            pltpu.SemaphoreType.DMA,
            pltpu.SemaphoreType.DMA,
        ],
    )(_body)
    rtau = jnp.full((16,), 1.0, jnp.float32) / tau_s
    out = run(x, log_w, rtau)
    return out.reshape(N, 1)


# speedup vs baseline: 886.0266x; 1.0039x over previous
import functools

import jax
import jax.numpy as jnp
from jax import lax
from jax.experimental import pallas as pl
from jax.experimental.pallas import tpu as pltpu
from jax.experimental.pallas import tpu_sc as plsc

N = 16384
C = 512
ROW = 2 * C
NW = 32
ROWS_PER_W = N // NW
CHUNK = 16
NCHUNK = ROWS_PER_W // CHUNK
TBL = 64 * 64


def _body(x_hbm, lw_hbm, rtau_hbm, out_hbm,
          lw_v, rtau_v, tab_v, red_v, xbuf0, xbuf1, out_v, sem0, sem1):
    nc = 2
    wid = lax.axis_index("s") * nc + lax.axis_index("c")
    row0 = wid * ROWS_PER_W

    pltpu.sync_copy(lw_hbm, lw_v)
    pltpu.sync_copy(rtau_hbm, rtau_v)
    rtau = rtau_v[...]

    lane = lax.iota(jnp.int32, 16)
    lane16 = lane * 16
    lane17 = lane * 17
    splats = [jnp.full((16,), k, jnp.int32) for k in range(16)]

    @pl.loop(0, TBL // 16)
    def _build(i):
        base = i * 16
        idx = base + lane
        a = idx >> 6
        b = idx & 63
        d = (a - b) & 127
        lw = plsc.load_gather(lw_v, [d])
        t = jnp.maximum(a, b).astype(jnp.float32)
        val = jnp.exp(lw - (2.0 - t) * rtau)
        for k in range(16):
            tab_v[pl.ds((base + k) * 16, 16)] = jnp.take(val, splats[k])

    def phase(ci, buf, sem):
        src = x_hbm.at[pl.ds(row0 + ci * CHUNK, CHUNK)]
        pltpu.make_async_copy(src, buf, sem).wait()

        @pl.loop(0, CHUNK)
        def _rows(r):

            @pl.loop(0, C // 16,
                     init_carry=jnp.zeros((16,), jnp.float32), unroll=8)
            def _inner(cc, acc):
                v0 = buf[r, 0, pl.ds(cc * 16, 16)]
                v1 = buf[r, 1, pl.ds(cc * 16, 16)]
                a = ((1.0 - v0) * 63.0).astype(jnp.int32)
                b = ((1.0 - v1) * 63.0).astype(jnp.int32)
                j = ((a << 10) | (b << 4)) | lane
                return acc + plsc.load_gather(tab_v, [j])

            red_v[pl.ds(r * 17, 16)] = _inner

        tot = jnp.zeros((16,), jnp.float32)
        for l in range(16):
            tot = tot + plsc.load_gather(red_v, [lane17 + l])
        out_v[pl.ds(ci * CHUNK, 16)] = tot

        @pl.when(ci + 2 < NCHUNK)
        def _():
            nsrc = x_hbm.at[pl.ds(row0 + (ci + 2) * CHUNK, CHUNK)]
            pltpu.async_copy(nsrc, buf, sem)

    pltpu.async_copy(x_hbm.at[pl.ds(row0, CHUNK)], xbuf0, sem0)
    pltpu.async_copy(x_hbm.at[pl.ds(row0 + CHUNK, CHUNK)], xbuf1, sem1)

    @pl.loop(0, NCHUNK, step=2)
    def _chunks(i):
        phase(i, xbuf0, sem0)
        phase(i + 1, xbuf1, sem1)

    pltpu.sync_copy(out_v, out_hbm.at[pl.ds(row0, ROWS_PER_W)])


@jax.jit
def kernel(x, log_w, tau_s):
    mesh = plsc.VectorSubcoreMesh(core_axis_name="c", subcore_axis_name="s")
    run = functools.partial(
        pl.kernel,
        mesh=mesh,
        compiler_params=pltpu.CompilerParams(needs_layout_passes=False),
        out_type=jax.ShapeDtypeStruct((N,), jnp.float32),
        scratch_types=[
            pltpu.VMEM((128,), jnp.float32),
            pltpu.VMEM((16,), jnp.float32),
            pltpu.VMEM((TBL * 16,), jnp.float32),
            pltpu.VMEM((CHUNK * 17,), jnp.float32),
            pltpu.VMEM((CHUNK, 2, C), jnp.float32),
            pltpu.VMEM((CHUNK, 2, C), jnp.float32),
            pltpu.VMEM((ROWS_PER_W,), jnp.float32),
            pltpu.SemaphoreType.DMA,
            pltpu.SemaphoreType.DMA,
        ],
    )(_body)
    rtau = jnp.full((16,), 1.0, jnp.float32) / tau_s
    out = run(x, log_w, rtau)
    return out.reshape(N, 1)
